# Initial kernel scaffold; baseline (speedup 1.0000x reference)
import jax
import jax.numpy as jnp
from jax.experimental import pallas as pl


def kernel(x, edge_index, W, b):
    raise NotImplementedError("write your pallas kernel here")


# speedup vs baseline: 7.8829x; 7.8829x over previous
import functools

import jax
import jax.numpy as jnp
from jax import lax
from jax.experimental import pallas as pl
from jax.experimental.pallas import tpu as pltpu
from jax.experimental.pallas import tpu_sc as plsc

N_NODES = 10000
N_EDGES = 320000
D = 128

NC = 2
NS = 16
NW = NC * NS
EPW = N_EDGES // NW
CHUNK = 100
NCHUNK = EPW // CHUNK
NPAD = 10240
RPT = NPAD // NS


def _sc_body(x_hbm, src_hbm, dst_hbm, zeros_hbm, out_hbm,
             acc, src_v, dst_v, rows_v, sem):
    c = lax.axis_index("c")
    s = lax.axis_index("s")
    wid = s * NC + c

    pltpu.sync_copy(zeros_hbm.at[pl.ds(s * RPT, RPT)],
                    acc.at[pl.ds(s * RPT, RPT)])
    pltpu.sync_copy(src_hbm.at[wid], src_v)
    pltpu.sync_copy(dst_hbm.at[wid], dst_v)
    plsc.subcore_barrier()

    def body(g, carry):
        pltpu.async_copy(x_hbm.at[src_v.at[g]], rows_v, sem).wait()
        pltpu.sync_copy(rows_v, acc.at[dst_v.at[g]], add=True)
        return carry

    lax.fori_loop(0, NCHUNK, body, 0, unroll=False)

    plsc.subcore_barrier()
    pltpu.sync_copy(acc.at[pl.ds(s * RPT, RPT)],
                    out_hbm.at[c, pl.ds(s * RPT, RPT)])


@functools.partial(
    pl.kernel,
    out_type=jax.ShapeDtypeStruct((NC, NPAD, D), jnp.float32),
    mesh=plsc.VectorSubcoreMesh(core_axis_name="c", subcore_axis_name="s"),
    scratch_types=[
        pltpu.VMEM_SHARED((NPAD, D), jnp.float32),
        pltpu.VMEM((NCHUNK, CHUNK), jnp.int32),
        pltpu.VMEM((NCHUNK, CHUNK), jnp.int32),
        pltpu.VMEM((CHUNK, D), jnp.float32),
        pltpu.SemaphoreType.DMA,
    ],
)
def _sc_pool(x_hbm, src_hbm, dst_hbm, zeros_hbm, out_hbm,
             acc, src_v, dst_v, rows_v, sem):
    _sc_body(x_hbm, src_hbm, dst_hbm, zeros_hbm, out_hbm,
             acc, src_v, dst_v, rows_v, sem)


def _tc_dense_body(x_ref, p0_ref, p1_ref, w_ref, b_ref, o_ref):
    pooled = p0_ref[...] + p1_ref[...]
    h = (jnp.dot(x_ref[...], w_ref[:D, :], preferred_element_type=jnp.float32)
         + jnp.dot(pooled, w_ref[D:, :], preferred_element_type=jnp.float32)
         + b_ref[...])
    o_ref[...] = jnp.maximum(h, 0.0)


def _tc_dense(x, p0, p1, W, b2):
    blk = 1000
    grid = (N_NODES // blk,)
    return pl.pallas_call(
        _tc_dense_body,
        grid=grid,
        in_specs=[
            pl.BlockSpec((blk, D), lambda i: (i, 0)),
            pl.BlockSpec((blk, D), lambda i: (i, 0)),
            pl.BlockSpec((blk, D), lambda i: (i, 0)),
            pl.BlockSpec((2 * D, D), lambda i: (0, 0)),
            pl.BlockSpec((1, D), lambda i: (0, 0)),
        ],
        out_specs=pl.BlockSpec((blk, D), lambda i: (i, 0)),
        out_shape=jax.ShapeDtypeStruct((N_NODES, D), jnp.float32),
    )(x, p0, p1, W, b2)


def kernel(x, edge_index, W, b):
    ei = edge_index.astype(jnp.int32)
    src3 = ei[0].reshape(NW, NCHUNK, CHUNK)
    dst3 = ei[1].reshape(NW, NCHUNK, CHUNK)
    zeros = jnp.zeros((NPAD, D), jnp.float32)
    partials = _sc_pool(x, src3, dst3, zeros)
    p0 = partials[0, :N_NODES]
    p1 = partials[1, :N_NODES]
    return _tc_dense(x, p0, p1, W, b.reshape(1, D))


# speedup vs baseline: 10.2194x; 1.2964x over previous
import functools

import jax
import jax.numpy as jnp
from jax import lax
from jax.experimental import pallas as pl
from jax.experimental.pallas import tpu as pltpu
from jax.experimental.pallas import tpu_sc as plsc

N_NODES = 10000
N_EDGES = 320000
D = 128

NC = 2
NS = 16
NW = NC * NS
EPW = N_EDGES // NW
CHUNK = 128
NCH = -(-EPW // CHUNK)
PADW = NCH * CHUNK - EPW
NPAD = 10240
RPT = NPAD // NS


def _sc_body(x_hbm, idx_hbm, zeros_hbm, out_hbm,
             acc, idxb0, idxb1, rows0, rows1, sg0, sg1, si0, si1):
    c = lax.axis_index("c")
    s = lax.axis_index("s")
    wid = s * NC + c

    pltpu.sync_copy(zeros_hbm.at[pl.ds(s * RPT, RPT)],
                    acc.at[pl.ds(s * RPT, RPT)])
    plsc.subcore_barrier()

    pltpu.sync_copy(idx_hbm.at[wid, 0], idxb0)
    pltpu.async_copy(x_hbm.at[idxb0.at[0]], rows0, sg0)
    pltpu.async_copy(idx_hbm.at[wid, 1], idxb1, si1)

    def body(h, carry):
        g0 = 2 * h
        g1 = g0 + 1
        pltpu.make_async_copy(idx_hbm.at[wid, g1], idxb1, si1).wait()
        pltpu.make_async_copy(x_hbm.at[idxb0.at[0]], rows0, sg0).wait()
        pltpu.async_copy(x_hbm.at[idxb1.at[0]], rows1, sg1)
        pltpu.sync_copy(rows0, acc.at[idxb0.at[1]], add=True)

        @pl.when(g0 + 2 < NCH)
        def _():
            pltpu.async_copy(idx_hbm.at[wid, g0 + 2], idxb0, si0)

        pltpu.make_async_copy(x_hbm.at[idxb1.at[0]], rows1, sg1).wait()

        @pl.when(g0 + 2 < NCH)
        def _():
            pltpu.make_async_copy(idx_hbm.at[wid, g0 + 2], idxb0, si0).wait()
            pltpu.async_copy(x_hbm.at[idxb0.at[0]], rows0, sg0)

        pltpu.sync_copy(rows1, acc.at[idxb1.at[1]], add=True)

        @pl.when(g1 + 2 < NCH)
        def _():
            pltpu.async_copy(idx_hbm.at[wid, g1 + 2], idxb1, si1)

        return carry

    lax.fori_loop(0, NCH // 2, body, 0, unroll=False)

    if NCH % 2:
        pltpu.make_async_copy(x_hbm.at[idxb0.at[0]], rows0, sg0).wait()
        pltpu.sync_copy(rows0, acc.at[idxb0.at[1]], add=True)

    plsc.subcore_barrier()
    pltpu.sync_copy(acc.at[pl.ds(s * RPT, RPT)],
                    out_hbm.at[c, pl.ds(s * RPT, RPT)])


@functools.partial(
    pl.kernel,
    out_type=jax.ShapeDtypeStruct((NC, NPAD, D), jnp.float32),
    mesh=plsc.VectorSubcoreMesh(core_axis_name="c", subcore_axis_name="s"),
    scratch_types=[
        pltpu.VMEM_SHARED((NPAD, D), jnp.float32),
        pltpu.VMEM((2, CHUNK), jnp.int32),
        pltpu.VMEM((2, CHUNK), jnp.int32),
        pltpu.VMEM((CHUNK, D), jnp.float32),
        pltpu.VMEM((CHUNK, D), jnp.float32),
        pltpu.SemaphoreType.DMA,
        pltpu.SemaphoreType.DMA,
        pltpu.SemaphoreType.DMA,
        pltpu.SemaphoreType.DMA,
    ],
)
def _sc_pool(x_hbm, idx_hbm, zeros_hbm, out_hbm,
             acc, idxb0, idxb1, rows0, rows1, sg0, sg1, si0, si1):
    _sc_body(x_hbm, idx_hbm, zeros_hbm, out_hbm,
             acc, idxb0, idxb1, rows0, rows1, sg0, sg1, si0, si1)


def _tc_dense_body(x_ref, p0_ref, p1_ref, w_ref, b_ref, o_ref):
    pooled = p0_ref[...] + p1_ref[...]
    h = (jnp.dot(x_ref[...], w_ref[:D, :], preferred_element_type=jnp.float32)
         + jnp.dot(pooled, w_ref[D:, :], preferred_element_type=jnp.float32)
         + b_ref[...])
    o_ref[...] = jnp.maximum(h, 0.0)


def _tc_dense(x, p0, p1, W, b2):
    blk = 1000
    grid = (N_NODES // blk,)
    return pl.pallas_call(
        _tc_dense_body,
        grid=grid,
        in_specs=[
            pl.BlockSpec((blk, D), lambda i: (i, 0)),
            pl.BlockSpec((blk, D), lambda i: (i, 0)),
            pl.BlockSpec((blk, D), lambda i: (i, 0)),
            pl.BlockSpec((2 * D, D), lambda i: (0, 0)),
            pl.BlockSpec((1, D), lambda i: (0, 0)),
        ],
        out_specs=pl.BlockSpec((blk, D), lambda i: (i, 0)),
        out_shape=jax.ShapeDtypeStruct((N_NODES, D), jnp.float32),
    )(x, p0, p1, W, b2)


def kernel(x, edge_index, W, b):
    ei = edge_index.astype(jnp.int32)
    s2 = ei[0].reshape(NW, EPW)
    d2 = ei[1].reshape(NW, EPW)
    lane = jnp.arange(PADW, dtype=jnp.int32)
    pad_s = jnp.broadcast_to((lane * 89) % N_NODES, (NW, PADW))
    pad_d = jnp.broadcast_to(N_NODES + (lane * 7) % (NPAD - N_NODES),
                             (NW, PADW))
    s3 = jnp.concatenate([s2, pad_s], axis=1).reshape(NW, NCH, CHUNK)
    d3 = jnp.concatenate([d2, pad_d], axis=1).reshape(NW, NCH, CHUNK)
    idx = jnp.stack([s3, d3], axis=2)
    zeros = jnp.zeros((NPAD, D), jnp.float32)
    partials = _sc_pool(x, idx, zeros)
    p0 = partials[0, :N_NODES]
    p1 = partials[1, :N_NODES]
    return _tc_dense(x, p0, p1, W, b.reshape(1, D))


# speedup vs baseline: 10.5913x; 1.0364x over previous
import functools

import jax
import jax.numpy as jnp
from jax import lax
from jax.experimental import pallas as pl
from jax.experimental.pallas import tpu as pltpu
from jax.experimental.pallas import tpu_sc as plsc

N_NODES = 10000
N_EDGES = 320000
D = 128

NC = 2
NS = 16
NW = NC * NS
EPW = N_EDGES // NW
CHUNK = 128
NFULL = EPW // CHUNK
NCH = NFULL + 1
TAILR = EPW - NFULL * CHUNK
NPAD = 10240
RPT = NPAD // NS


def _sc_body(x_hbm, src_hbm, dst_hbm, tail_hbm, zeros_hbm, out_hbm,
             acc, idxb0, idxb1, rows0, rows1,
             sg0, sg1, ss0, sd0, ss1, sd1):
    c = lax.axis_index("c")
    s = lax.axis_index("s")
    wid = s * NC + c

    pltpu.sync_copy(zeros_hbm.at[pl.ds(s * RPT, RPT)],
                    acc.at[pl.ds(s * RPT, RPT)])
    plsc.subcore_barrier()

    def start_idx(g, buf, sem_s, sem_d):
        @pl.when(g < NFULL)
        def _():
            base = wid * EPW + g * CHUNK
            pltpu.async_copy(src_hbm.at[pl.ds(base, CHUNK)], buf.at[0], sem_s)
            pltpu.async_copy(dst_hbm.at[pl.ds(base, CHUNK)], buf.at[1], sem_d)

        @pl.when(g >= NFULL)
        def _():
            pltpu.async_copy(tail_hbm.at[wid, 0], buf.at[0], sem_s)
            pltpu.async_copy(tail_hbm.at[wid, 1], buf.at[1], sem_d)

    def wait_idx(buf, sem_s, sem_d):
        pltpu.make_async_copy(src_hbm.at[pl.ds(0, CHUNK)], buf.at[0],
                              sem_s).wait()
        pltpu.make_async_copy(dst_hbm.at[pl.ds(0, CHUNK)], buf.at[1],
                              sem_d).wait()

    start_idx(0, idxb0, ss0, sd0)
    wait_idx(idxb0, ss0, sd0)
    pltpu.async_copy(x_hbm.at[idxb0.at[0]], rows0, sg0)
    start_idx(1, idxb1, ss1, sd1)

    def body(h, carry):
        g0 = 2 * h

        wait_idx(idxb1, ss1, sd1)
        pltpu.make_async_copy(x_hbm.at[idxb0.at[0]], rows0, sg0).wait()
        pltpu.async_copy(x_hbm.at[idxb1.at[0]], rows1, sg1)
        pltpu.sync_copy(rows0, acc.at[idxb0.at[1]], add=True)

        @pl.when(g0 + 2 < NCH)
        def _():
            start_idx(g0 + 2, idxb0, ss0, sd0)

        pltpu.make_async_copy(x_hbm.at[idxb1.at[0]], rows1, sg1).wait()

        @pl.when(g0 + 2 < NCH)
        def _():
            wait_idx(idxb0, ss0, sd0)
            pltpu.async_copy(x_hbm.at[idxb0.at[0]], rows0, sg0)

        pltpu.sync_copy(rows1, acc.at[idxb1.at[1]], add=True)

        @pl.when(g0 + 3 < NCH)
        def _():
            start_idx(g0 + 3, idxb1, ss1, sd1)

        return carry

    lax.fori_loop(0, NCH // 2, body, 0, unroll=False)

    if NCH % 2:
        pltpu.make_async_copy(x_hbm.at[idxb0.at[0]], rows0, sg0).wait()
        pltpu.sync_copy(rows0, acc.at[idxb0.at[1]], add=True)

    plsc.subcore_barrier()
    pltpu.sync_copy(acc.at[pl.ds(s * RPT, RPT)],
                    out_hbm.at[c, pl.ds(s * RPT, RPT)])


@functools.partial(
    pl.kernel,
    out_type=jax.ShapeDtypeStruct((NC, NPAD, D), jnp.float32),
    mesh=plsc.VectorSubcoreMesh(core_axis_name="c", subcore_axis_name="s"),
    scratch_types=[
        pltpu.VMEM_SHARED((NPAD, D), jnp.float32),
        pltpu.VMEM((2, CHUNK), jnp.int32),
        pltpu.VMEM((2, CHUNK), jnp.int32),
        pltpu.VMEM((CHUNK, D), jnp.float32),
        pltpu.VMEM((CHUNK, D), jnp.float32),
        pltpu.SemaphoreType.DMA,
        pltpu.SemaphoreType.DMA,
        pltpu.SemaphoreType.DMA,
        pltpu.SemaphoreType.DMA,
        pltpu.SemaphoreType.DMA,
        pltpu.SemaphoreType.DMA,
    ],
)
def _sc_pool(x_hbm, src_hbm, dst_hbm, tail_hbm, zeros_hbm, out_hbm,
             acc, idxb0, idxb1, rows0, rows1, sg0, sg1, ss0, sd0, ss1, sd1):
    _sc_body(x_hbm, src_hbm, dst_hbm, tail_hbm, zeros_hbm, out_hbm,
             acc, idxb0, idxb1, rows0, rows1, sg0, sg1, ss0, sd0, ss1, sd1)


def _tc_dense_body(x_ref, p0_ref, p1_ref, w_ref, b_ref, o_ref):
    pooled = p0_ref[...] + p1_ref[...]
    h = (jnp.dot(x_ref[...], w_ref[:D, :], preferred_element_type=jnp.float32)
         + jnp.dot(pooled, w_ref[D:, :], preferred_element_type=jnp.float32)
         + b_ref[...])
    o_ref[...] = jnp.maximum(h, 0.0)


def _tc_dense(x, p0, p1, W, b2):
    blk = 1000
    grid = (N_NODES // blk,)
    return pl.pallas_call(
        _tc_dense_body,
        grid=grid,
        in_specs=[
            pl.BlockSpec((blk, D), lambda i: (i, 0)),
            pl.BlockSpec((blk, D), lambda i: (i, 0)),
            pl.BlockSpec((blk, D), lambda i: (i, 0)),
            pl.BlockSpec((2 * D, D), lambda i: (0, 0)),
            pl.BlockSpec((1, D), lambda i: (0, 0)),
        ],
        out_specs=pl.BlockSpec((blk, D), lambda i: (i, 0)),
        out_shape=jax.ShapeDtypeStruct((N_NODES, D), jnp.float32),
    )(x, p0, p1, W, b2)


def kernel(x, edge_index, W, b):
    ei = edge_index.astype(jnp.int32)
    src = ei[0]
    dst = ei[1]
    lane = jnp.arange(CHUNK - TAILR, dtype=jnp.int32)
    pad_s = jnp.broadcast_to((lane * 89) % N_NODES, (NW, CHUNK - TAILR))
    pad_d = jnp.broadcast_to(N_NODES + (lane * 7) % (NPAD - N_NODES),
                             (NW, CHUNK - TAILR))
    s_tail = jnp.concatenate(
        [src.reshape(NW, EPW)[:, NFULL * CHUNK:], pad_s], axis=1)
    d_tail = jnp.concatenate(
        [dst.reshape(NW, EPW)[:, NFULL * CHUNK:], pad_d], axis=1)
    tail = jnp.stack([s_tail, d_tail], axis=1)
    zeros = jnp.zeros((NPAD, D), jnp.float32)
    partials = _sc_pool(x, src, dst, tail, zeros)
    p0 = partials[0, :N_NODES]
    p1 = partials[1, :N_NODES]
    return _tc_dense(x, p0, p1, W, b.reshape(1, D))


# speedup vs baseline: 12.0090x; 1.1339x over previous
import functools

import jax
import jax.numpy as jnp
from jax import lax
from jax.experimental import pallas as pl
from jax.experimental.pallas import tpu as pltpu
from jax.experimental.pallas import tpu_sc as plsc

N_NODES = 10000
N_EDGES = 320000
D = 128

NC = 2
NS = 16
NW = NC * NS
EPW = N_EDGES // NW
CHUNK = 112
NFULL = EPW // CHUNK
NCH = NFULL + 1
TAILR = EPW - NFULL * CHUNK
NPAD = 10240
RPT = NPAD // NS
R = 3


def _sc_body(x_hbm, src_hbm, dst_hbm, tail_hbm, zeros_hbm, out_hbm,
             acc, idxb, rows, si, sg, sc):
    c = lax.axis_index("c")
    s = lax.axis_index("s")
    wid = s * NC + c

    pltpu.sync_copy(zeros_hbm.at[pl.ds(s * RPT, RPT)],
                    acc.at[pl.ds(s * RPT, RPT)])
    plsc.subcore_barrier()

    def start_idx(g, b):
        @pl.when(g < NFULL)
        def _():
            base = wid * EPW + g * CHUNK
            pltpu.async_copy(src_hbm.at[pl.ds(base, CHUNK)],
                             idxb[b].at[0], si[b])
            pltpu.async_copy(dst_hbm.at[pl.ds(base, CHUNK)],
                             idxb[b].at[1], si[b])

        @pl.when(g >= NFULL)
        def _():
            pltpu.async_copy(tail_hbm.at[wid, 0], idxb[b].at[0], si[b])
            pltpu.async_copy(tail_hbm.at[wid, 1], idxb[b].at[1], si[b])

    def wait_idx(b):
        pltpu.make_async_copy(src_hbm.at[pl.ds(0, CHUNK)], idxb[b].at[0],
                              si[b]).wait()
        pltpu.make_async_copy(dst_hbm.at[pl.ds(0, CHUNK)], idxb[b].at[1],
                              si[b]).wait()

    def start_gather(b):
        pltpu.async_copy(x_hbm.at[idxb[b].at[0]], rows[b], sg[b])

    def wait_gather(b):
        pltpu.make_async_copy(x_hbm.at[idxb[b].at[0]], rows[b], sg[b]).wait()

    def start_scatter(b):
        pltpu.async_copy(rows[b], acc.at[idxb[b].at[1]], sc[b], add=True)

    def wait_scatter(b):
        pltpu.make_async_copy(rows[b], acc.at[idxb[b].at[1]], sc[b]).wait()

    start_idx(0, 0)
    start_idx(1, 1)
    wait_idx(0)
    start_gather(0)
    start_idx(2, 2)
    wait_idx(1)
    start_gather(1)
    wait_gather(0)
    start_scatter(0)

    def body(k, carry):
        g0 = R * k
        for j in range(R):
            b2 = (j - 2) % R
            wait_scatter(j)
            start_idx(g0 + j, j)
            wait_idx((j - 1) % R)
            start_gather((j - 1) % R)
            wait_gather(b2)
            start_scatter(b2)
        return carry

    lax.fori_loop(1, NCH // R, body, 0, unroll=False)

    wait_scatter(0)
    wait_idx(2)
    start_gather(2)
    wait_gather(1)
    start_scatter(1)
    wait_scatter(1)
    wait_gather(2)
    start_scatter(2)
    wait_scatter(2)

    plsc.subcore_barrier()
    pltpu.sync_copy(acc.at[pl.ds(s * RPT, RPT)],
                    out_hbm.at[c, pl.ds(s * RPT, RPT)])


@functools.partial(
    pl.kernel,
    out_type=jax.ShapeDtypeStruct((NC, NPAD, D), jnp.float32),
    mesh=plsc.VectorSubcoreMesh(core_axis_name="c", subcore_axis_name="s"),
    scratch_types=[
        pltpu.VMEM_SHARED((NPAD, D), jnp.float32),
        [pltpu.VMEM((2, CHUNK), jnp.int32) for _ in range(R)],
        [pltpu.VMEM((CHUNK, D), jnp.float32) for _ in range(R)],
        [pltpu.SemaphoreType.DMA for _ in range(R)],
        [pltpu.SemaphoreType.DMA for _ in range(R)],
        [pltpu.SemaphoreType.DMA for _ in range(R)],
    ],
)
def _sc_pool(x_hbm, src_hbm, dst_hbm, tail_hbm, zeros_hbm, out_hbm,
             acc, idxb, rows, si, sg, sc):
    _sc_body(x_hbm, src_hbm, dst_hbm, tail_hbm, zeros_hbm, out_hbm,
             acc, idxb, rows, si, sg, sc)


def _tc_dense_body(x_ref, p0_ref, p1_ref, w_ref, b_ref, o_ref):
    pooled = p0_ref[...] + p1_ref[...]
    h = (jnp.dot(x_ref[...], w_ref[:D, :], preferred_element_type=jnp.float32)
         + jnp.dot(pooled, w_ref[D:, :], preferred_element_type=jnp.float32)
         + b_ref[...])
    o_ref[...] = jnp.maximum(h, 0.0)


def _tc_dense(x, p0, p1, W, b2):
    blk = 1000
    grid = (N_NODES // blk,)
    return pl.pallas_call(
        _tc_dense_body,
        grid=grid,
        in_specs=[
            pl.BlockSpec((blk, D), lambda i: (i, 0)),
            pl.BlockSpec((blk, D), lambda i: (i, 0)),
            pl.BlockSpec((blk, D), lambda i: (i, 0)),
            pl.BlockSpec((2 * D, D), lambda i: (0, 0)),
            pl.BlockSpec((1, D), lambda i: (0, 0)),
        ],
        out_specs=pl.BlockSpec((blk, D), lambda i: (i, 0)),
        out_shape=jax.ShapeDtypeStruct((N_NODES, D), jnp.float32),
    )(x, p0, p1, W, b2)


def kernel(x, edge_index, W, b):
    ei = edge_index.astype(jnp.int32)
    src = ei[0]
    dst = ei[1]
    lane = jnp.arange(CHUNK - TAILR, dtype=jnp.int32)
    pad_s = jnp.broadcast_to((lane * 89) % N_NODES, (NW, CHUNK - TAILR))
    pad_d = jnp.broadcast_to(N_NODES + (lane * 7) % (NPAD - N_NODES),
                             (NW, CHUNK - TAILR))
    s_tail = jnp.concatenate(
        [src.reshape(NW, EPW)[:, NFULL * CHUNK:], pad_s], axis=1)
    d_tail = jnp.concatenate(
        [dst.reshape(NW, EPW)[:, NFULL * CHUNK:], pad_d], axis=1)
    tail = jnp.stack([s_tail, d_tail], axis=1)
    zeros = jnp.zeros((NPAD, D), jnp.float32)
    partials = _sc_pool(x, src, dst, tail, zeros)
    p0 = partials[0, :N_NODES]
    p1 = partials[1, :N_NODES]
    return _tc_dense(x, p0, p1, W, b.reshape(1, D))
